# Initial kernel scaffold; baseline (speedup 1.0000x reference)
import jax
import jax.numpy as jnp
from jax.experimental import pallas as pl


def kernel(x_microbes, x_diseases, edge_index_m2d, edge_index_d2m, edge_label_index, W1_l_m2d, b1_m2d, W1_r_m2d, W1_l_d2m, b1_d2m, W1_r_d2m, W2_l_m2d, b2_m2d, W2_r_m2d, W2_l_d2m, b2_d2m, W2_r_d2m, Wd1, bd1, Wd2, bd2):
    raise NotImplementedError("write your pallas kernel here")


# speedup vs baseline: 4.4325x; 4.4325x over previous
import jax
import jax.numpy as jnp
from jax import lax
from jax.experimental import pallas as pl
from jax.experimental.pallas import tpu as pltpu, tpu_sc as plsc

N_M = 10000
N_D = 10000
E = 320000
D = 128
L = 100000

NC = 2
NS = 16
NW = NC * NS
CHUNK = 128
CH_E = 80
CH_H = CH_E // 2
EPAD = NW * CH_E * CHUNK
NDP = 10240
RPT = NDP // NS
CH_L = 25
LPAD = NW * CH_L * CHUNK

_MESH = dict(core_axis_name="c", subcore_axis_name="s",
             num_cores=NC, num_subcores=NS)


def _seg_sum_body(x_hbm, isrc_hbm, idst_hbm, zrow_hbm, sums_out,
                  idx_s, idx_d, rows, sem, acc):
    c = lax.axis_index("c")
    s = lax.axis_index("s")
    wid = s * NC + c
    pltpu.sync_copy(zrow_hbm, rows)
    for r in range(RPT // CHUNK):
        pltpu.sync_copy(rows, acc.at[pl.ds(s * RPT + r * CHUNK, CHUNK)])
    plsc.subcore_barrier()

    def body(j, carry):
        pltpu.async_copy(x_hbm.at[idx_s.at[j]], rows, sem).wait()
        pltpu.sync_copy(rows, acc.at[idx_d.at[j]], add=True)
        return carry

    for h in range(2):
        pltpu.sync_copy(isrc_hbm.at[wid * 2 + h], idx_s)
        pltpu.sync_copy(idst_hbm.at[wid * 2 + h], idx_d)
        lax.fori_loop(0, CH_H, body, 0)
    plsc.subcore_barrier()
    base = c * NDP + s * RPT
    for r in range(RPT // CHUNK):
        pltpu.sync_copy(acc.at[pl.ds(s * RPT + r * CHUNK, CHUNK)], rows)
        pltpu.sync_copy(rows, sums_out.at[pl.ds(base + r * CHUNK, CHUNK)])


_seg_sum = pl.kernel(
    _seg_sum_body,
    out_type=(jax.ShapeDtypeStruct((NC * NDP, D), jnp.float32),),
    mesh=plsc.VectorSubcoreMesh(**_MESH),
    scratch_types=[
        pltpu.VMEM((CH_H, CHUNK), jnp.int32),
        pltpu.VMEM((CH_H, CHUNK), jnp.int32),
        pltpu.VMEM((CHUNK, D), jnp.float32),
        pltpu.SemaphoreType.DMA,
        pltpu.VMEM_SHARED((NDP, D), jnp.float32),
    ],
    name="seg_sum_sc")


def _gather_body(zm_hbm, zd_hbm, im_hbm, id_hbm, gm_out, gd_out,
                 idx_v, rows, sem):
    c = lax.axis_index("c")
    s = lax.axis_index("s")
    wid = s * NC + c
    for z_hbm, i_hbm, out in ((zm_hbm, im_hbm, gm_out), (zd_hbm, id_hbm, gd_out)):
        pltpu.sync_copy(i_hbm.at[wid], idx_v)

        def body(j, carry):
            pltpu.async_copy(z_hbm.at[idx_v.at[j]], rows, sem).wait()
            pltpu.sync_copy(
                rows, out.at[pl.ds(wid * (CH_L * CHUNK) + j * CHUNK, CHUNK)])
            return carry

        lax.fori_loop(0, CH_L, body, 0)


_gather_pairs = pl.kernel(
    _gather_body,
    out_type=(jax.ShapeDtypeStruct((LPAD, D), jnp.float32),
              jax.ShapeDtypeStruct((LPAD, D), jnp.float32)),
    mesh=plsc.VectorSubcoreMesh(**_MESH),
    scratch_types=[
        pltpu.VMEM((CH_L, CHUNK), jnp.int32),
        pltpu.VMEM((CHUNK, D), jnp.float32),
        pltpu.SemaphoreType.DMA,
    ],
    name="gather_pairs_sc")


_RB = 1000


def _sage_dense_relu_body(p_ref, c_ref, x_ref, wl_ref, wr_ref, b_ref, o_ref):
    _sage_dense_common(True, p_ref, c_ref, x_ref, wl_ref, wr_ref, b_ref, o_ref)


def _sage_dense_lin_body(p_ref, c_ref, x_ref, wl_ref, wr_ref, b_ref, o_ref):
    _sage_dense_common(False, p_ref, c_ref, x_ref, wl_ref, wr_ref, b_ref, o_ref)


def _sage_dense_common(relu, p_ref, c_ref, x_ref, wl_ref, wr_ref, b_ref, o_ref):
    ps = p_ref[0] + p_ref[1]
    cnt = (c_ref[0] + c_ref[1])[:, 0:1]
    mean = ps / jnp.maximum(cnt, 1.0)
    acc = jnp.dot(mean, wl_ref[...], preferred_element_type=jnp.float32)
    acc = acc + jnp.dot(x_ref[...], wr_ref[...], preferred_element_type=jnp.float32)
    acc = acc + b_ref[...]
    o_ref[...] = jnp.maximum(acc, 0.0) if relu else acc


def _sage_dense(p, cnt, x, w_l, b_l, w_r, relu):
    n = x.shape[0]
    grid = n // _RB
    return pl.pallas_call(
        _sage_dense_relu_body if relu else _sage_dense_lin_body,
        grid=(grid,),
        in_specs=[
            pl.BlockSpec((NC, _RB, D), lambda i: (0, i, 0)),
            pl.BlockSpec((NC, _RB, D), lambda i: (0, i, 0)),
            pl.BlockSpec((_RB, D), lambda i: (i, 0)),
            pl.BlockSpec((D, D), lambda i: (0, 0)),
            pl.BlockSpec((D, D), lambda i: (0, 0)),
            pl.BlockSpec((1, D), lambda i: (0, 0)),
        ],
        out_specs=pl.BlockSpec((_RB, D), lambda i: (i, 0)),
        out_shape=jax.ShapeDtypeStruct((n, D), jnp.float32),
    )(p, cnt, x, w_l, w_r, b_l)


_DB = 2048


def _decoder_body(gm_ref, gd_ref, w1_ref, b1_ref, w2_ref, b2_ref, o_ref):
    z = jnp.dot(gm_ref[...], w1_ref[0:D, :], preferred_element_type=jnp.float32)
    z = z + jnp.dot(gd_ref[...], w1_ref[D:2 * D, :],
                    preferred_element_type=jnp.float32)
    z = jnp.maximum(z + b1_ref[...], 0.0)
    o_ref[...] = jnp.dot(z, w2_ref[...], preferred_element_type=jnp.float32) \
        + b2_ref[...]


def _decoder(gm, gd, w1, b1, w2, b2):
    grid = LPAD // _DB
    return pl.pallas_call(
        _decoder_body,
        grid=(grid,),
        in_specs=[
            pl.BlockSpec((_DB, D), lambda i: (i, 0)),
            pl.BlockSpec((_DB, D), lambda i: (i, 0)),
            pl.BlockSpec((2 * D, D), lambda i: (0, 0)),
            pl.BlockSpec((1, D), lambda i: (0, 0)),
            pl.BlockSpec((D, 1), lambda i: (0, 0)),
            pl.BlockSpec((1, 1), lambda i: (0, 0)),
        ],
        out_specs=pl.BlockSpec((_DB, 1), lambda i: (i, 0)),
        out_shape=jax.ShapeDtypeStruct((LPAD, 1), jnp.float32),
    )(gm, gd, w1, b1, w2, b2)


def _pad_edge_idx(idx, n_pad, pad_val_base):
    npad = n_pad - idx.shape[0]
    pad = pad_val_base + (jnp.arange(npad, dtype=jnp.int32) % (NDP - N_D))
    return jnp.concatenate([idx.astype(jnp.int32), pad]).reshape(NW, -1, CHUNK)


def kernel(x_microbes, x_diseases, edge_index_m2d, edge_index_d2m,
           edge_label_index,
           W1_l_m2d, b1_m2d, W1_r_m2d, W1_l_d2m, b1_d2m, W1_r_d2m,
           W2_l_m2d, b2_m2d, W2_r_m2d, W2_l_d2m, b2_d2m, W2_r_d2m,
           Wd1, bd1, Wd2, bd2):
    zrow = jnp.zeros((CHUNK, D), jnp.float32)
    ones_tbl = jnp.ones((N_M, D), jnp.float32)

    src_m2d = _pad_edge_idx(edge_index_m2d[0], EPAD, 0).reshape(NW * 2, CH_H, CHUNK)
    dst_m2d = _pad_edge_idx(edge_index_m2d[1], EPAD, N_D).reshape(NW * 2, CH_H, CHUNK)
    src_d2m = _pad_edge_idx(edge_index_d2m[0], EPAD, 0).reshape(NW * 2, CH_H, CHUNK)
    dst_d2m = _pad_edge_idx(edge_index_d2m[1], EPAD, N_M).reshape(NW * 2, CH_H, CHUNK)
    row_lbl = _pad_edge_idx(edge_label_index[0], LPAD, 0)
    col_lbl = _pad_edge_idx(edge_label_index[1], LPAD, 0)

    b1_m2d_2 = b1_m2d.reshape(1, D)
    b1_d2m_2 = b1_d2m.reshape(1, D)
    b2_m2d_2 = b2_m2d.reshape(1, D)
    b2_d2m_2 = b2_d2m.reshape(1, D)

    (p_d,) = _seg_sum(x_microbes, src_m2d, dst_m2d, zrow)
    (p_m,) = _seg_sum(x_diseases, src_d2m, dst_d2m, zrow)
    (c_d,) = _seg_sum(ones_tbl, src_m2d, dst_m2d, zrow)
    (c_m,) = _seg_sum(ones_tbl, src_d2m, dst_d2m, zrow)
    p_d = p_d.reshape(NC, NDP, D)
    p_m = p_m.reshape(NC, NDP, D)
    c_d = c_d.reshape(NC, NDP, D)
    c_m = c_m.reshape(NC, NDP, D)
    h_d = _sage_dense(p_d, c_d, x_diseases, W1_l_m2d, b1_m2d_2, W1_r_m2d, True)
    h_m = _sage_dense(p_m, c_m, x_microbes, W1_l_d2m, b1_d2m_2, W1_r_d2m, True)

    (q_d,) = _seg_sum(h_m, src_m2d, dst_m2d, zrow)
    (q_m,) = _seg_sum(h_d, src_d2m, dst_d2m, zrow)
    q_d = q_d.reshape(NC, NDP, D)
    q_m = q_m.reshape(NC, NDP, D)
    z_d = _sage_dense(q_d, c_d, h_d, W2_l_m2d, b2_m2d_2, W2_r_m2d, False)
    z_m = _sage_dense(q_m, c_m, h_m, W2_l_d2m, b2_d2m_2, W2_r_d2m, False)

    g_m, g_d = _gather_pairs(z_m, z_d, row_lbl, col_lbl)
    out = _decoder(g_m, g_d, Wd1, bd1.reshape(1, D), Wd2, bd2.reshape(1, 1))
    return out[:L]


# speedup vs baseline: 5.2215x; 1.1780x over previous
import jax
import jax.numpy as jnp
from jax import lax
from jax.experimental import pallas as pl
from jax.experimental.pallas import tpu as pltpu, tpu_sc as plsc

N_M = 10000
N_D = 10000
E = 320000
D = 128
L = 100000

NC = 2
NS = 16
NW = NC * NS
CHUNK = 128
CH_E = 80
CH_H = CH_E // 2
EPAD = NW * CH_E * CHUNK
NDP = 10240
RPT = NDP // NS
CH_L = 25
LPAD = NW * CH_L * CHUNK

_MESH = dict(core_axis_name="c", subcore_axis_name="s",
             num_cores=NC, num_subcores=NS)


def _seg_sum_body(x_hbm, isrc_hbm, idst_hbm, zrow_hbm, sums_out,
                  idx_s, idx_d, rows, sem, acc):
    c = lax.axis_index("c")
    s = lax.axis_index("s")
    wid = s * NC + c
    pltpu.sync_copy(zrow_hbm, rows)
    for r in range(RPT // CHUNK):
        pltpu.sync_copy(rows, acc.at[pl.ds(s * RPT + r * CHUNK, CHUNK)])
    plsc.subcore_barrier()

    def body(j, carry):
        pltpu.async_copy(x_hbm.at[idx_s.at[j]], rows, sem).wait()
        pltpu.sync_copy(rows, acc.at[idx_d.at[j]], add=True)
        return carry

    for h in range(2):
        pltpu.sync_copy(isrc_hbm.at[wid * 2 + h], idx_s)
        pltpu.sync_copy(idst_hbm.at[wid * 2 + h], idx_d)
        lax.fori_loop(0, CH_H, body, 0)
    plsc.subcore_barrier()
    base = c * NDP + s * RPT
    for r in range(RPT // CHUNK):
        pltpu.sync_copy(acc.at[pl.ds(s * RPT + r * CHUNK, CHUNK)], rows)
        pltpu.sync_copy(rows, sums_out.at[pl.ds(base + r * CHUNK, CHUNK)])


_seg_sum = pl.kernel(
    _seg_sum_body,
    out_type=(jax.ShapeDtypeStruct((NC * NDP, D), jnp.float32),),
    mesh=plsc.VectorSubcoreMesh(**_MESH),
    scratch_types=[
        pltpu.VMEM((CH_H, CHUNK), jnp.int32),
        pltpu.VMEM((CH_H, CHUNK), jnp.int32),
        pltpu.VMEM((CHUNK, D), jnp.float32),
        pltpu.SemaphoreType.DMA,
        pltpu.VMEM_SHARED((NDP, D), jnp.float32),
    ],
    name="seg_sum_sc")


def _cnt_body(idst_hbm, ones_hbm, zrow_hbm, cnts_out, idx_d, rows, acc):
    c = lax.axis_index("c")
    s = lax.axis_index("s")
    wid = s * NC + c
    pltpu.sync_copy(zrow_hbm, rows)
    for r in range(RPT // CHUNK):
        pltpu.sync_copy(rows, acc.at[pl.ds(s * RPT + r * CHUNK, CHUNK)])
    pltpu.sync_copy(ones_hbm, rows)
    plsc.subcore_barrier()

    def body(j, carry):
        pltpu.sync_copy(rows, acc.at[idx_d.at[j]], add=True)
        return carry

    for h in range(2):
        pltpu.sync_copy(idst_hbm.at[wid * 2 + h], idx_d)
        lax.fori_loop(0, CH_H, body, 0)
    plsc.subcore_barrier()
    base = c * NDP + s * RPT
    for r in range(RPT // CHUNK):
        pltpu.sync_copy(acc.at[pl.ds(s * RPT + r * CHUNK, CHUNK)], rows)
        pltpu.sync_copy(rows, cnts_out.at[pl.ds(base + r * CHUNK, CHUNK)])


_cnt_sum = pl.kernel(
    _cnt_body,
    out_type=(jax.ShapeDtypeStruct((NC * NDP, D), jnp.float32),),
    mesh=plsc.VectorSubcoreMesh(**_MESH),
    scratch_types=[
        pltpu.VMEM((CH_H, CHUNK), jnp.int32),
        pltpu.VMEM((CHUNK, D), jnp.float32),
        pltpu.VMEM_SHARED((NDP, D), jnp.float32),
    ],
    name="cnt_sum_sc")


def _gather_body(zm_hbm, zd_hbm, im_hbm, id_hbm, gm_out, gd_out,
                 idx_v, rows, sem):
    c = lax.axis_index("c")
    s = lax.axis_index("s")
    wid = s * NC + c
    for z_hbm, i_hbm, out in ((zm_hbm, im_hbm, gm_out), (zd_hbm, id_hbm, gd_out)):
        pltpu.sync_copy(i_hbm.at[wid], idx_v)

        def body(j, carry):
            pltpu.async_copy(z_hbm.at[idx_v.at[j]], rows, sem).wait()
            pltpu.sync_copy(
                rows, out.at[pl.ds(wid * (CH_L * CHUNK) + j * CHUNK, CHUNK)])
            return carry

        lax.fori_loop(0, CH_L, body, 0)


_gather_pairs = pl.kernel(
    _gather_body,
    out_type=(jax.ShapeDtypeStruct((LPAD, D), jnp.float32),
              jax.ShapeDtypeStruct((LPAD, D), jnp.float32)),
    mesh=plsc.VectorSubcoreMesh(**_MESH),
    scratch_types=[
        pltpu.VMEM((CH_L, CHUNK), jnp.int32),
        pltpu.VMEM((CHUNK, D), jnp.float32),
        pltpu.SemaphoreType.DMA,
    ],
    name="gather_pairs_sc")


_RB = 1000


def _sage_dense_relu_body(p_ref, c_ref, x_ref, wl_ref, wr_ref, b_ref, o_ref):
    _sage_dense_common(True, p_ref, c_ref, x_ref, wl_ref, wr_ref, b_ref, o_ref)


def _sage_dense_lin_body(p_ref, c_ref, x_ref, wl_ref, wr_ref, b_ref, o_ref):
    _sage_dense_common(False, p_ref, c_ref, x_ref, wl_ref, wr_ref, b_ref, o_ref)


def _sage_dense_common(relu, p_ref, c_ref, x_ref, wl_ref, wr_ref, b_ref, o_ref):
    ps = p_ref[0] + p_ref[1]
    cnt = (c_ref[0] + c_ref[1])[:, 0:1]
    mean = ps / jnp.maximum(cnt, 1.0)
    acc = jnp.dot(mean, wl_ref[...], preferred_element_type=jnp.float32)
    acc = acc + jnp.dot(x_ref[...], wr_ref[...], preferred_element_type=jnp.float32)
    acc = acc + b_ref[...]
    o_ref[...] = jnp.maximum(acc, 0.0) if relu else acc


def _sage_dense(p, cnt, x, w_l, b_l, w_r, relu):
    n = x.shape[0]
    grid = n // _RB
    return pl.pallas_call(
        _sage_dense_relu_body if relu else _sage_dense_lin_body,
        grid=(grid,),
        in_specs=[
            pl.BlockSpec((NC, _RB, D), lambda i: (0, i, 0)),
            pl.BlockSpec((NC, _RB, D), lambda i: (0, i, 0)),
            pl.BlockSpec((_RB, D), lambda i: (i, 0)),
            pl.BlockSpec((D, D), lambda i: (0, 0)),
            pl.BlockSpec((D, D), lambda i: (0, 0)),
            pl.BlockSpec((1, D), lambda i: (0, 0)),
        ],
        out_specs=pl.BlockSpec((_RB, D), lambda i: (i, 0)),
        out_shape=jax.ShapeDtypeStruct((n, D), jnp.float32),
    )(p, cnt, x, w_l, w_r, b_l)


_DB = 2048


def _decoder_body(gm_ref, gd_ref, w1_ref, b1_ref, w2_ref, b2_ref, o_ref):
    z = jnp.dot(gm_ref[...], w1_ref[0:D, :], preferred_element_type=jnp.float32)
    z = z + jnp.dot(gd_ref[...], w1_ref[D:2 * D, :],
                    preferred_element_type=jnp.float32)
    z = jnp.maximum(z + b1_ref[...], 0.0)
    o_ref[...] = jnp.dot(z, w2_ref[...], preferred_element_type=jnp.float32) \
        + b2_ref[...]


def _decoder(gm, gd, w1, b1, w2, b2):
    grid = LPAD // _DB
    return pl.pallas_call(
        _decoder_body,
        grid=(grid,),
        in_specs=[
            pl.BlockSpec((_DB, D), lambda i: (i, 0)),
            pl.BlockSpec((_DB, D), lambda i: (i, 0)),
            pl.BlockSpec((2 * D, D), lambda i: (0, 0)),
            pl.BlockSpec((1, D), lambda i: (0, 0)),
            pl.BlockSpec((D, 1), lambda i: (0, 0)),
            pl.BlockSpec((1, 1), lambda i: (0, 0)),
        ],
        out_specs=pl.BlockSpec((_DB, 1), lambda i: (i, 0)),
        out_shape=jax.ShapeDtypeStruct((LPAD, 1), jnp.float32),
    )(gm, gd, w1, b1, w2, b2)


def _pad_edge_idx(idx, n_pad, pad_val_base):
    npad = n_pad - idx.shape[0]
    pad = pad_val_base + (jnp.arange(npad, dtype=jnp.int32) % (NDP - N_D))
    return jnp.concatenate([idx.astype(jnp.int32), pad]).reshape(NW, -1, CHUNK)


def kernel(x_microbes, x_diseases, edge_index_m2d, edge_index_d2m,
           edge_label_index,
           W1_l_m2d, b1_m2d, W1_r_m2d, W1_l_d2m, b1_d2m, W1_r_d2m,
           W2_l_m2d, b2_m2d, W2_r_m2d, W2_l_d2m, b2_d2m, W2_r_d2m,
           Wd1, bd1, Wd2, bd2):
    zrow = jnp.zeros((CHUNK, D), jnp.float32)
    ones_blk = jnp.ones((CHUNK, D), jnp.float32)

    src_m2d = _pad_edge_idx(edge_index_m2d[0], EPAD, 0).reshape(NW * 2, CH_H, CHUNK)
    dst_m2d = _pad_edge_idx(edge_index_m2d[1], EPAD, N_D).reshape(NW * 2, CH_H, CHUNK)
    src_d2m = _pad_edge_idx(edge_index_d2m[0], EPAD, 0).reshape(NW * 2, CH_H, CHUNK)
    dst_d2m = _pad_edge_idx(edge_index_d2m[1], EPAD, N_M).reshape(NW * 2, CH_H, CHUNK)
    row_lbl = _pad_edge_idx(edge_label_index[0], LPAD, 0)
    col_lbl = _pad_edge_idx(edge_label_index[1], LPAD, 0)

    b1_m2d_2 = b1_m2d.reshape(1, D)
    b1_d2m_2 = b1_d2m.reshape(1, D)
    b2_m2d_2 = b2_m2d.reshape(1, D)
    b2_d2m_2 = b2_d2m.reshape(1, D)

    (p_d,) = _seg_sum(x_microbes, src_m2d, dst_m2d, zrow)
    (p_m,) = _seg_sum(x_diseases, src_d2m, dst_d2m, zrow)
    (c_d,) = _cnt_sum(dst_m2d, ones_blk, zrow)
    (c_m,) = _cnt_sum(dst_d2m, ones_blk, zrow)
    p_d = p_d.reshape(NC, NDP, D)
    p_m = p_m.reshape(NC, NDP, D)
    c_d = c_d.reshape(NC, NDP, D)
    c_m = c_m.reshape(NC, NDP, D)
    h_d = _sage_dense(p_d, c_d, x_diseases, W1_l_m2d, b1_m2d_2, W1_r_m2d, True)
    h_m = _sage_dense(p_m, c_m, x_microbes, W1_l_d2m, b1_d2m_2, W1_r_d2m, True)

    (q_d,) = _seg_sum(h_m, src_m2d, dst_m2d, zrow)
    (q_m,) = _seg_sum(h_d, src_d2m, dst_d2m, zrow)
    q_d = q_d.reshape(NC, NDP, D)
    q_m = q_m.reshape(NC, NDP, D)
    z_d = _sage_dense(q_d, c_d, h_d, W2_l_m2d, b2_m2d_2, W2_r_m2d, False)
    z_m = _sage_dense(q_m, c_m, h_m, W2_l_d2m, b2_d2m_2, W2_r_d2m, False)

    g_m, g_d = _gather_pairs(z_m, z_d, row_lbl, col_lbl)
    out = _decoder(g_m, g_d, Wd1, bd1.reshape(1, D), Wd2, bd2.reshape(1, 1))
    return out[:L]
